# baseline (speedup 1.0000x reference)
import functools

import jax
import jax.numpy as jnp
from jax import lax
from jax.experimental import pallas as pl
from jax.experimental.pallas import tpu as pltpu
from jax.experimental.pallas import tpu_sc as plsc

_N = 2048
_D = 768
_NB = 8
_BLK = 256
_AUG = 128
_DA = _D + _AUG
_EPS = 1e-5
_IND4 = 1.0 / (_D ** 0.25)
_D4 = _D ** 0.25
_SHIFT = 64.0 + 1.0 + 28.0
_MU = 28.0
_NW = 32
_RPW = _N // _NW


def _prep_body(x_ref, mask_ref, rot_ref, xa_ref, rank_ref, offs_ref):
    x = x_ref[...]
    mu = jnp.mean(x, axis=1, keepdims=True)
    var = jnp.mean((x - mu) ** 2, axis=1, keepdims=True)
    xn = (x - mu) * lax.rsqrt(var + _EPS)
    xm = xn * mask_ref[...]
    s4 = jnp.dot(xn, rot_ref[...], preferred_element_type=jnp.float32)
    s = jnp.concatenate([s4, -s4], axis=1)
    smax = jnp.max(s, axis=1, keepdims=True)
    idx8 = lax.broadcasted_iota(jnp.int32, s.shape, 1)
    cand = jnp.where(s == smax, idx8, _NB)
    first = jnp.min(cand, axis=1, keepdims=True)
    oh = (idx8 == first).astype(jnp.float32)
    idx128 = lax.broadcasted_iota(jnp.int32, (_N, _AUG), 1)
    aug = (jnp.where(idx128 == first, 8.0, 0.0)
           + jnp.where(idx128 == _NB, 1.0, 0.0))
    xa_ref[...] = jnp.concatenate([xm * _IND4, aug], axis=1)

    ir = lax.broadcasted_iota(jnp.int32, (_N, _N), 0)
    ic = lax.broadcasted_iota(jnp.int32, (_N, _N), 1)
    ltri = (ic < ir).astype(jnp.bfloat16)
    prefix = jnp.dot(ltri, oh.astype(jnp.bfloat16),
                     preferred_element_type=jnp.float32)
    cnt = jnp.sum(oh, axis=0, keepdims=True)
    br = lax.broadcasted_iota(jnp.int32, (_NB, _NB), 0)
    bc = lax.broadcasted_iota(jnp.int32, (_NB, _NB), 1)
    ut = (br < bc).astype(jnp.float32)
    offs = jnp.dot(cnt, ut, preferred_element_type=jnp.float32)
    rank = jnp.sum(oh * (prefix + offs), axis=1, keepdims=True)
    rank_ref[...] = rank.astype(jnp.int32)[:, 0]
    offs_ref[...] = jnp.concatenate(
        [offs, jnp.full((1, _NB), float(_N), jnp.float32)],
        axis=1).astype(jnp.int32)


def _sc_scatter_body(xa_hbm, rank_hbm, xs_hbm, idx_v, rows_v, sem):
    wid = lax.axis_index("s") * 2 + lax.axis_index("c")
    base = wid * _RPW
    pltpu.sync_copy(rank_hbm.at[pl.ds(base, _RPW)], idx_v)
    pltpu.sync_copy(xa_hbm.at[pl.ds(base, _RPW)], rows_v)
    pltpu.async_copy(rows_v, xs_hbm.at[idx_v], sem).wait()


def _sc_unsort_body(osort_hbm, rank_hbm, out_hbm, idx_v, rows_v, sem):
    wid = lax.axis_index("s") * 2 + lax.axis_index("c")
    base = wid * _RPW
    pltpu.sync_copy(rank_hbm.at[pl.ds(base, _RPW)], idx_v)
    pltpu.async_copy(osort_hbm.at[idx_v], rows_v, sem).wait()
    pltpu.sync_copy(rows_v, out_hbm.at[pl.ds(base, _RPW)])


def _attn_body(offs_ref, xs_ref, out_ref, xb_s):
    i = pl.program_id(0)

    @pl.when(i == 0)
    def _cvt():
        xb_s[...] = xs_ref[...].astype(jnp.bfloat16)

    @pl.when(i > 0)
    def _attn():
        r0 = (i - 1) * _BLK
        start = jnp.int32(0)
        for c in range(1, _NB):
            oc = offs_ref[c]
            start = jnp.where(oc <= r0, oc, start)
        end = jnp.int32(_N)
        for c in range(_NB - 1, 0, -1):
            oc = offs_ref[c]
            end = jnp.where(oc > r0 + _BLK - 1, oc, end)
        jlo = start // _BLK
        jhi = (end + _BLK - 1) // _BLK

        qa = xb_s[pl.ds(r0, _BLK), :]

        def col_step(j, acc):
            ka = xb_s[pl.ds(j * _BLK, _BLK), :]
            s = lax.dot_general(qa, ka, (((1,), (1,)), ((), ())),
                                preferred_element_type=jnp.float32)
            p = jnp.exp(s - _SHIFT)
            return acc + lax.dot_general(p.astype(jnp.bfloat16), ka,
                                         (((1,), (0,)), ((), ())),
                                         preferred_element_type=jnp.float32)

        acc = lax.fori_loop(jlo, jhi, col_step,
                            jnp.zeros((_BLK, _DA), jnp.float32))
        l = acc[:, _D + _NB:_D + _NB + 1]
        riota = lax.broadcasted_iota(jnp.int32, (_BLK, 1), 0) + r0
        b = jnp.zeros((_BLK, 1), jnp.int32)
        for c in range(1, _NB):
            b = b + jnp.where(offs_ref[c] <= riota, 1, 0)
        cnt = jnp.zeros((_BLK, 1), jnp.int32)
        for c in range(_NB):
            sz = (jnp.int32(_N) if c == _NB - 1 else offs_ref[c + 1]) - offs_ref[c]
            cnt = jnp.where(b == c, sz, cnt)
        den = l + (_N - cnt).astype(jnp.float32) * jnp.exp(-_MU)
        out_ref[...] = acc[:, :_D] * (_D4 / den)


@jax.jit
def kernel(x, input_mask, rotations):
    x2 = x[0]
    mask2 = input_mask[0][:, None]
    rot = rotations[0, :, -1, :]

    xa, rank, offs2d = pl.pallas_call(
        _prep_body,
        out_shape=(
            jax.ShapeDtypeStruct((_N, _DA), jnp.float32),
            jax.ShapeDtypeStruct((_N,), jnp.int32),
            jax.ShapeDtypeStruct((1, 16), jnp.int32),
        ),
    )(x2, mask2, rot)
    offs = offs2d.reshape(16)

    mesh = plsc.VectorSubcoreMesh(core_axis_name="c", subcore_axis_name="s")
    scatter = functools.partial(
        pl.kernel,
        mesh=mesh,
        out_type=jax.ShapeDtypeStruct((_N, _DA), jnp.float32),
        scratch_types=[
            pltpu.VMEM((_RPW,), jnp.int32),
            pltpu.VMEM((_RPW, _DA), jnp.float32),
            pltpu.SemaphoreType.DMA,
        ],
    )(_sc_scatter_body)
    xs = scatter(xa, rank)

    out_sorted = pl.pallas_call(
        _attn_body,
        grid_spec=pltpu.PrefetchScalarGridSpec(
            num_scalar_prefetch=1,
            grid=(_N // _BLK + 1,),
            in_specs=[pl.BlockSpec((_N, _DA), lambda i, o: (0, 0))],
            out_specs=pl.BlockSpec(
                (_BLK, _D), lambda i, o: (jnp.maximum(i - 1, 0), 0)),
            scratch_shapes=[pltpu.VMEM((_N, _DA), jnp.bfloat16)],
        ),
        out_shape=jax.ShapeDtypeStruct((_N, _D), jnp.float32),
    )(offs, xs)

    unsort = functools.partial(
        pl.kernel,
        mesh=mesh,
        out_type=jax.ShapeDtypeStruct((_N, _D), jnp.float32),
        scratch_types=[
            pltpu.VMEM((_RPW,), jnp.int32),
            pltpu.VMEM((_RPW, _D), jnp.float32),
            pltpu.SemaphoreType.DMA,
        ],
    )(_sc_unsort_body)
    out = unsort(out_sorted, rank)

    return out[None]


# speedup vs baseline: 2.5772x; 2.5772x over previous
import jax
import jax.numpy as jnp
from jax import lax
from jax.experimental import pallas as pl
from jax.experimental.pallas import tpu as pltpu

_N = 2048
_D = 768
_NB = 8
_BLK = 1024
_AUG = 128
_DA = _D + _AUG
_EPS = 1e-5
_IND4 = 1.0 / (_D ** 0.25)
_D4 = _D ** 0.25
_SHIFT = 64.0 + 1.0 + 28.0
_MU = 28.0


def _body(x_ref, mask_ref, rot_ref, out_ref, xa_s, z_s):
    i = pl.program_id(0)

    @pl.when(i == 0)
    def _prep():
        x = x_ref[...]
        mu = jnp.mean(x, axis=1, keepdims=True)
        var = jnp.mean((x - mu) ** 2, axis=1, keepdims=True)
        xn = (x - mu) * lax.rsqrt(var + _EPS)
        xm = xn * mask_ref[...]
        rot = rot_ref[...]
        s4 = jnp.dot(xn, rot, preferred_element_type=jnp.float32)
        s = jnp.concatenate([s4, -s4], axis=1)
        smax = jnp.max(s, axis=1, keepdims=True)
        idx8 = lax.broadcasted_iota(jnp.int32, s.shape, 1)
        cand = jnp.where(s == smax, idx8, _NB)
        first = jnp.min(cand, axis=1, keepdims=True)
        oh = (idx8 == first).astype(jnp.float32)
        cnt = jnp.sum(oh, axis=0, keepdims=True)
        z_s[...] = float(_N) - lax.dot_general(
            oh, cnt, (((1,), (1,)), ((), ())),
            preferred_element_type=jnp.float32)
        idx128 = lax.broadcasted_iota(jnp.int32, (_N, _AUG), 1)
        aug = (jnp.where(idx128 == first, 8.0, 0.0)
               + jnp.where(idx128 == _NB, 1.0, 0.0))
        xa_s[...] = jnp.concatenate(
            [xm * _IND4, aug], axis=1).astype(jnp.bfloat16)

    @pl.when(i > 0)
    def _attn():
        r0 = (i - 1) * _BLK
        qa = xa_s[pl.ds(r0, _BLK), :]
        xa = xa_s[...]
        s = lax.dot_general(qa, xa, (((1,), (1,)), ((), ())),
                            preferred_element_type=jnp.float32)
        p = jnp.exp(s - _SHIFT)
        acc = lax.dot_general(p.astype(jnp.bfloat16), xa,
                              (((1,), (0,)), ((), ())),
                              preferred_element_type=jnp.float32)
        l = acc[:, _D + _NB:_D + _NB + 1]
        den = l + z_s[pl.ds(r0, _BLK), :] * jnp.exp(-_MU)
        out_ref[...] = acc[:, :_D] * (_D4 / den)


@jax.jit
def kernel(x, input_mask, rotations):
    x2 = x[0]
    mask2 = input_mask[0][:, None]
    rot = rotations[0, :, -1, :]
    nblk = _N // _BLK
    out = pl.pallas_call(
        _body,
        grid=(nblk + 1,),
        in_specs=[
            pl.BlockSpec((_N, _D), lambda i: (0, 0)),
            pl.BlockSpec((_N, 1), lambda i: (0, 0)),
            pl.BlockSpec((_D, _NB // 2), lambda i: (0, 0)),
        ],
        out_specs=pl.BlockSpec((_BLK, _D),
                               lambda i: (jnp.maximum(i - 1, 0), 0)),
        out_shape=jax.ShapeDtypeStruct((_N, _D), jnp.float32),
        scratch_shapes=[
            pltpu.VMEM((_N, _DA), jnp.bfloat16),
            pltpu.VMEM((_N, 1), jnp.float32),
        ],
    )(x2, mask2, rot)

    return out[None]


# speedup vs baseline: 2.7739x; 1.0763x over previous
import jax
import jax.numpy as jnp
from jax import lax
from jax.experimental import pallas as pl
from jax.experimental.pallas import tpu as pltpu

_N = 2048
_D = 768
_NB = 8
_BLK = 1024
_AUG = 128
_DA = _D + _AUG
_EPS = 1e-5
_IND4 = 1.0 / (_D ** 0.25)
_D4 = _D ** 0.25
_SHIFT = 64.0 + 1.0 + 28.0
_MU = 28.0


def _body(x_ref, mask_ref, rot_ref, out_ref, xa_s, z_s):
    i = pl.program_id(0)

    @pl.when(i == 0)
    def _prep():
        x = x_ref[...]
        mu = jnp.mean(x, axis=1, keepdims=True)
        var = jnp.mean((x - mu) ** 2, axis=1, keepdims=True)
        xn = (x - mu) * lax.rsqrt(var + _EPS)
        xm = xn * mask_ref[...]
        rot = rot_ref[...]
        s4 = jnp.dot(xn, rot, preferred_element_type=jnp.float32)
        s = jnp.concatenate([s4, -s4], axis=1)
        smax = jnp.max(s, axis=1, keepdims=True)
        idx8 = lax.broadcasted_iota(jnp.int32, s.shape, 1)
        cand = jnp.where(s == smax, idx8, _NB)
        first = jnp.min(cand, axis=1, keepdims=True)
        oh = (idx8 == first).astype(jnp.float32)
        cnt = jnp.sum(oh, axis=0, keepdims=True)
        z_s[...] = float(_N) - lax.dot_general(
            oh, cnt, (((1,), (1,)), ((), ())),
            preferred_element_type=jnp.float32)
        idx128 = lax.broadcasted_iota(jnp.int32, (_N, _AUG), 1)
        aug = (jnp.where(idx128 == first, 8.0, 0.0)
               + jnp.where(idx128 == _NB, 1.0, 0.0))
        xa_s[...] = jnp.concatenate(
            [xm * _IND4, aug], axis=1).astype(jnp.bfloat16)

    @pl.when(i > 0)
    def _attn():
        r0 = (i - 1) * _BLK
        qa = xa_s[pl.ds(r0, _BLK), :]
        xa = xa_s[...]
        s = lax.dot_general(qa, xa, (((1,), (1,)), ((), ())),
                            preferred_element_type=jnp.float32)
        p = jnp.exp(s - _SHIFT)
        acc = lax.dot_general(p.astype(jnp.bfloat16), xa[:, :_D],
                              (((1,), (0,)), ((), ())),
                              preferred_element_type=jnp.float32)
        l = jnp.sum(p, axis=1, keepdims=True)
        den = l + z_s[pl.ds(r0, _BLK), :] * jnp.exp(-_MU)
        out_ref[...] = acc * (_D4 / den)


@jax.jit
def kernel(x, input_mask, rotations):
    x2 = x[0]
    mask2 = input_mask[0][:, None]
    rot = rotations[0, :, -1, :]
    nblk = _N // _BLK
    out = pl.pallas_call(
        _body,
        grid=(nblk + 1,),
        in_specs=[
            pl.BlockSpec((_N, _D), lambda i: (0, 0)),
            pl.BlockSpec((_N, 1), lambda i: (0, 0)),
            pl.BlockSpec((_D, _NB // 2), lambda i: (0, 0)),
        ],
        out_specs=pl.BlockSpec((_BLK, _D),
                               lambda i: (jnp.maximum(i - 1, 0), 0)),
        out_shape=jax.ShapeDtypeStruct((_N, _D), jnp.float32),
        scratch_shapes=[
            pltpu.VMEM((_N, _DA), jnp.bfloat16),
            pltpu.VMEM((_N, 1), jnp.float32),
        ],
    )(x2, mask2, rot)

    return out[None]
